# baseline (speedup 1.0000x reference)
import functools

import jax
import jax.numpy as jnp
from jax import lax
from jax.experimental import pallas as pl
from jax.experimental.pallas import tpu as pltpu
from jax.experimental.pallas import tpu_sc as plsc

RANK = 64
GATHER_W = 128
BLK = 512
NP_PAD = 1024


def _sc_gather(entity_table, s_idx, o_idx):
    b = s_idx.shape[1]
    mesh = plsc.VectorSubcoreMesh(core_axis_name="core", subcore_axis_name="subcore")
    out_t = (
        jax.ShapeDtypeStruct((b, RANK), jnp.float32),
        jax.ShapeDtypeStruct((b, RANK), jnp.float32),
    )

    @functools.partial(
        pl.kernel, out_type=out_t, mesh=mesh,
        compiler_params=pltpu.CompilerParams(use_tc_tiling_on_sc=False),
    )
    def gather_kernel(tab_hbm, si_hbm, oi_hbm, so_hbm, oo_hbm):
        def body(si_v, oi_v, so_v, oo_v):
            pltpu.sync_copy(tab_hbm.at[si_v.at[0]], so_v)
            pltpu.sync_copy(tab_hbm.at[oi_v.at[0]], oo_v)

        pltpu.emit_pipeline(
            body,
            grid=(b // GATHER_W,),
            in_specs=[
                pl.BlockSpec((1, GATHER_W), lambda i: (0, i)),
                pl.BlockSpec((1, GATHER_W), lambda i: (0, i)),
            ],
            out_specs=[
                pl.BlockSpec((GATHER_W, RANK), lambda i: (i, 0)),
                pl.BlockSpec((GATHER_W, RANK), lambda i: (i, 0)),
            ],
            core_axis_name=("core", "subcore"),
            dimension_semantics=(pltpu.PARALLEL,),
        )(si_hbm, oi_hbm, so_hbm, oo_hbm)

    return gather_kernel(entity_table, s_idx, o_idx)


def _tc_body(s_ref, o_ref, p_ref, pt_ref, out_ref, x_ref):
    s = s_ref[...]
    o = o_ref[...]
    for k in range(RANK // 2):
        left = s[:, 2 * k : 2 * k + 1] * o
        right = s[:, 2 * k + 1 : 2 * k + 2] * o
        chunk = jnp.concatenate([left, right], axis=1)
        x_ref[:, 128 * k : 128 * (k + 1)] = chunk.astype(jnp.bfloat16)
    scores = lax.dot_general(
        x_ref[...], pt_ref[...],
        (((1,), (1,)), ((), ())),
        preferred_element_type=jnp.float32,
    )
    pidx = p_ref[0]
    sel = pidx == lax.broadcasted_iota(jnp.int32, (BLK, NP_PAD), 1)
    spo = jnp.sum(jnp.where(sel, scores, 0.0), axis=1, keepdims=True)
    out_ref[...] = jax.nn.sigmoid(spo)


def kernel(s_input, p_input, o_input, entity_table, predicate_table):
    b = s_input.shape[0]
    np_real = predicate_table.shape[0]
    s_idx = s_input.reshape(1, b).astype(jnp.int32)
    o_idx = o_input.reshape(1, b).astype(jnp.int32)
    s_emb, o_emb = _sc_gather(entity_table, s_idx, o_idx)

    pflat = predicate_table.reshape(np_real, RANK * RANK).astype(jnp.bfloat16)
    pflat = jnp.pad(pflat, ((0, NP_PAD - np_real), (0, 0)))
    p3 = p_input.reshape(b // BLK, BLK, 1).astype(jnp.int32)

    out = pl.pallas_call(
        _tc_body,
        grid=(b // BLK,),
        in_specs=[
            pl.BlockSpec((BLK, RANK), lambda i: (i, 0)),
            pl.BlockSpec((BLK, RANK), lambda i: (i, 0)),
            pl.BlockSpec((1, BLK, 1), lambda i: (i, 0, 0)),
            pl.BlockSpec((NP_PAD, RANK * RANK), lambda i: (0, 0)),
        ],
        out_specs=pl.BlockSpec((BLK, 1), lambda i: (i, 0)),
        out_shape=jax.ShapeDtypeStruct((b, 1), jnp.float32),
        scratch_shapes=[pltpu.VMEM((BLK, RANK * RANK), jnp.bfloat16)],
    )(s_emb, o_emb, p3, pflat)
    return out


# speedup vs baseline: 1.3348x; 1.3348x over previous
import functools

import jax
import jax.numpy as jnp
from jax import lax
from jax.experimental import pallas as pl
from jax.experimental.pallas import tpu as pltpu
from jax.experimental.pallas import tpu_sc as plsc

RANK = 64
GATHER_W = 128
BLK = 512
NP_PAD = 1024


def _sc_gather(entity_table, s_idx, o_idx):
    b = s_idx.shape[0] * s_idx.shape[1]
    w = GATHER_W
    mesh = plsc.VectorSubcoreMesh(core_axis_name="core", subcore_axis_name="subcore")
    out_t = (
        jax.ShapeDtypeStruct((b, RANK), jnp.float32),
        jax.ShapeDtypeStruct((b, RANK), jnp.float32),
    )

    @functools.partial(
        pl.kernel, out_type=out_t, mesh=mesh,
        scratch_types=[pltpu.SemaphoreType.DMA],
    )
    def gather_kernel(tab_hbm, si_hbm, oi_hbm, so_hbm, oo_hbm, sem):
        def one_table(idx_v, out_v):
            cps = []
            for k in range(w // 16):
                iv = idx_v[0, pl.ds(16 * k, 16)]
                for jj in range(16):
                    j = 16 * k + jj
                    cps.append(pltpu.async_copy(
                        tab_hbm.at[pl.ds(iv[jj], 1)],
                        out_v.at[pl.ds(j, 1)],
                        sem,
                    ))
            for cp in cps:
                cp.wait()

        def body(si_v, oi_v, so_v, oo_v):
            one_table(si_v, so_v)
            one_table(oi_v, oo_v)

        pltpu.emit_pipeline(
            body,
            grid=(b // w,),
            in_specs=[
                pl.BlockSpec((1, w), lambda i: (i, 0)),
                pl.BlockSpec((1, w), lambda i: (i, 0)),
            ],
            out_specs=[
                pl.BlockSpec((w, RANK), lambda i: (i, 0)),
                pl.BlockSpec((w, RANK), lambda i: (i, 0)),
            ],
            core_axis_name=("core", "subcore"),
            dimension_semantics=(pltpu.PARALLEL,),
        )(si_hbm, oi_hbm, so_hbm, oo_hbm)

    return gather_kernel(entity_table, s_idx, o_idx)


def _tc_body(s_ref, o_ref, p_ref, pt_ref, out_ref, x_ref):
    s = s_ref[:, :RANK]
    o = o_ref[:, :RANK]
    for k in range(RANK // 2):
        left = s[:, 2 * k : 2 * k + 1] * o
        right = s[:, 2 * k + 1 : 2 * k + 2] * o
        chunk = jnp.concatenate([left, right], axis=1)
        x_ref[:, 128 * k : 128 * (k + 1)] = chunk.astype(jnp.bfloat16)
    scores = lax.dot_general(
        x_ref[...], pt_ref[...],
        (((1,), (1,)), ((), ())),
        preferred_element_type=jnp.float32,
    )
    pidx = p_ref[0]
    sel = pidx == lax.broadcasted_iota(jnp.int32, (BLK, NP_PAD), 1)
    spo = jnp.sum(jnp.where(sel, scores, 0.0), axis=1, keepdims=True)
    out_ref[...] = jax.nn.sigmoid(spo)


def kernel(s_input, p_input, o_input, entity_table, predicate_table):
    b = s_input.shape[0]
    np_real = predicate_table.shape[0]
    s_idx = s_input.reshape(b // GATHER_W, GATHER_W).astype(jnp.int32)
    o_idx = o_input.reshape(b // GATHER_W, GATHER_W).astype(jnp.int32)
    s_emb, o_emb = _sc_gather(entity_table, s_idx, o_idx)

    pflat = predicate_table.reshape(np_real, RANK * RANK).astype(jnp.bfloat16)
    pflat = jnp.pad(pflat, ((0, NP_PAD - np_real), (0, 0)))
    p3 = p_input.reshape(b // BLK, BLK, 1).astype(jnp.int32)

    out = pl.pallas_call(
        _tc_body,
        grid=(b // BLK,),
        in_specs=[
            pl.BlockSpec((BLK, RANK), lambda i: (i, 0)),
            pl.BlockSpec((BLK, RANK), lambda i: (i, 0)),
            pl.BlockSpec((1, BLK, 1), lambda i: (i, 0, 0)),
            pl.BlockSpec((NP_PAD, RANK * RANK), lambda i: (0, 0)),
        ],
        out_specs=pl.BlockSpec((BLK, 1), lambda i: (i, 0)),
        out_shape=jax.ShapeDtypeStruct((b, 1), jnp.float32),
        scratch_shapes=[pltpu.VMEM((BLK, RANK * RANK), jnp.bfloat16)],
    )(s_emb, o_emb, p3, pflat)
    return out


# speedup vs baseline: 1.5037x; 1.1265x over previous
import jax
import jax.numpy as jnp
from jax import lax
from jax.experimental import pallas as pl
from jax.experimental.pallas import tpu as pltpu

RANK = 64
BLK = 512
NP_PAD = 1024


def _issue(tab_ref, idx_ref, base, buf, slot, sem):
    def one(j, _):
        idx = idx_ref[base + j]
        pltpu.make_async_copy(
            tab_ref.at[pl.ds(idx, 1)],
            buf.at[slot, pl.ds(j, 1)],
            sem.at[slot],
        ).start()
        return 0

    lax.fori_loop(0, BLK, one, 0, unroll=8)


def _wait(buf, slot, sem):
    pltpu.make_async_copy(buf.at[slot], buf.at[slot], sem.at[slot]).wait()


def _body(si_ref, oi_ref, tab_ref, p_ref, pt_ref, sm_ref, tm_ref, out_ref,
          sbuf, obuf, sem_s, sem_o):
    i = pl.program_id(0)
    n = pl.num_programs(0)
    slot = lax.rem(i, 2)

    @pl.when(i == 0)
    def _prologue():
        _issue(tab_ref, si_ref, 0, sbuf, 0, sem_s)
        _issue(tab_ref, oi_ref, 0, obuf, 0, sem_o)

    @pl.when(i + 1 < n)
    def _prefetch_next():
        nxt = lax.rem(i + 1, 2)
        _issue(tab_ref, si_ref, (i + 1) * BLK, sbuf, nxt, sem_s)
        _issue(tab_ref, oi_ref, (i + 1) * BLK, obuf, nxt, sem_o)

    _wait(sbuf, slot, sem_s)
    _wait(obuf, slot, sem_o)

    s = sbuf[slot].astype(jnp.bfloat16)
    o = obuf[slot].astype(jnp.bfloat16)
    s_rep = lax.dot_general(s, sm_ref[...], (((1,), (0,)), ((), ())),
                            preferred_element_type=jnp.float32)
    o_tile = lax.dot_general(o, tm_ref[...], (((1,), (0,)), ((), ())),
                             preferred_element_type=jnp.float32)
    x = (s_rep * o_tile).astype(jnp.bfloat16)
    scores = lax.dot_general(x, pt_ref[...], (((1,), (0,)), ((), ())),
                             preferred_element_type=jnp.float32)
    pidx = p_ref[0]
    sel = pidx == lax.broadcasted_iota(jnp.int32, (BLK, NP_PAD), 1)
    spo = jnp.sum(jnp.where(sel, scores, 0.0), axis=1, keepdims=True)
    out_ref[...] = jax.nn.sigmoid(spo)


def kernel(s_input, p_input, o_input, entity_table, predicate_table):
    b = s_input.shape[0]
    np_real = predicate_table.shape[0]
    s_idx = s_input.reshape(b).astype(jnp.int32)
    o_idx = o_input.reshape(b).astype(jnp.int32)
    p3 = p_input.reshape(b // BLK, BLK, 1).astype(jnp.int32)

    ptt = predicate_table.reshape(np_real, RANK * RANK).astype(jnp.bfloat16)
    ptt = jnp.pad(ptt, ((0, NP_PAD - np_real), (0, 0))).T

    k = jnp.arange(RANK * RANK, dtype=jnp.int32)
    ar = jnp.arange(RANK, dtype=jnp.int32)
    sm = (ar[:, None] == k[None, :] // RANK).astype(jnp.bfloat16)
    tm = (ar[:, None] == k[None, :] % RANK).astype(jnp.bfloat16)

    grid_spec = pltpu.PrefetchScalarGridSpec(
        num_scalar_prefetch=2,
        grid=(b // BLK,),
        in_specs=[
            pl.BlockSpec(memory_space=pl.ANY),
            pl.BlockSpec((1, BLK, 1), lambda i, si, oi: (i, 0, 0)),
            pl.BlockSpec((RANK * RANK, NP_PAD), lambda i, si, oi: (0, 0)),
            pl.BlockSpec((RANK, RANK * RANK), lambda i, si, oi: (0, 0)),
            pl.BlockSpec((RANK, RANK * RANK), lambda i, si, oi: (0, 0)),
        ],
        out_specs=pl.BlockSpec((BLK, 1), lambda i, si, oi: (i, 0)),
        scratch_shapes=[
            pltpu.VMEM((2, BLK, RANK), jnp.float32),
            pltpu.VMEM((2, BLK, RANK), jnp.float32),
            pltpu.SemaphoreType.DMA((2,)),
            pltpu.SemaphoreType.DMA((2,)),
        ],
    )
    out = pl.pallas_call(
        _body,
        grid_spec=grid_spec,
        out_shape=jax.ShapeDtypeStruct((b, 1), jnp.float32),
    )(s_idx, o_idx, entity_table, p3, ptt, sm, tm)
    return out
